# baseline (speedup 1.0000x reference)
import functools

import jax
import jax.numpy as jnp
from jax import lax
from jax.experimental import pallas as pl
from jax.experimental.pallas import tpu as pltpu
from jax.experimental.pallas import tpu_sc as plsc

_N = 16384
_H = 256
_NG = 256
_ZMAX = 100
_ZPAD = 128
_B = 1024
_NB = _N // _B

_NC = 2
_NS = 16
_NW = _NC * _NS
_CHUNK = _N // _NW


def _head_body(z_ref, pos_ref, emb_ref, Wp_ref, Wv_ref, Wo1_ref,
               Wo2_ref, W1_ref, out_ref, embW1_ref):
    i = pl.program_id(0)

    @pl.when(i == 0)
    def _():
        embW1_ref[...] = jnp.dot(emb_ref[...], W1_ref[...],
                                 preferred_element_type=jnp.float32)

    z = z_ref[...]
    oh = (z == jax.lax.broadcasted_iota(jnp.int32, (_B, _ZPAD), 1)
          ).astype(jnp.float32)
    posb = pos_ref[...]
    x = jnp.dot(oh, embW1_ref[...], preferred_element_type=jnp.float32)
    x = x + jnp.dot(posb, Wp_ref[...], preferred_element_type=jnp.float32)
    x = x * jax.nn.sigmoid(x)
    u = jnp.dot(x, Wv_ref[...], preferred_element_type=jnp.float32)
    pos2 = jnp.sum(posb * posb, axis=1, keepdims=True)
    g = x + pos2 * (u * u)
    s = jnp.dot(g, Wo1_ref[...], preferred_element_type=jnp.float32)
    s = s * jax.nn.sigmoid(s)
    out_ref[...] = jnp.dot(s, Wo2_ref[...],
                           preferred_element_type=jnp.float32)


def _head(z2, pos_pad, emb_pad, Wp_pad, Wv, Wo1, Wo2, W1):
    return pl.pallas_call(
        _head_body,
        grid=(_NB,),
        in_specs=[
            pl.BlockSpec((_B, 1), lambda i: (i, 0)),
            pl.BlockSpec((_B, 8), lambda i: (i, 0)),
            pl.BlockSpec((_ZPAD, _H), lambda i: (0, 0)),
            pl.BlockSpec((8, _H), lambda i: (0, 0)),
            pl.BlockSpec((_H, _H), lambda i: (0, 0)),
            pl.BlockSpec((_H, _H // 2), lambda i: (0, 0)),
            pl.BlockSpec((_H // 2, 1), lambda i: (0, 0)),
            pl.BlockSpec((_H, _H), lambda i: (0, 0)),
        ],
        out_specs=pl.BlockSpec((_B, 1), lambda i: (i, 0)),
        out_shape=jax.ShapeDtypeStruct((_N, 1), jnp.float32),
        scratch_shapes=[pltpu.VMEM((_ZPAD, _H), jnp.float32)],
    )(z2, pos_pad, emb_pad, Wp_pad, Wv, Wo1, Wo2, W1)


def _segsum_body(pa_hbm, batch_hbm, idx0_hbm, out_hbm,
                 vals_v, ids_v, acc_v, idx0_v, shared):
    c = lax.axis_index("c")
    s = lax.axis_index("s")
    wid = s * _NC + c
    base = wid * _CHUNK
    pltpu.sync_copy(pa_hbm.at[pl.ds(base, _CHUNK)], vals_v)
    pltpu.sync_copy(batch_hbm.at[pl.ds(base, _CHUNK)], ids_v)
    pltpu.sync_copy(idx0_hbm, idx0_v)

    zero = jnp.zeros((16,), jnp.float32)
    for j in range(_NG // 16):
        acc_v[0, pl.ds(j * 16, 16)] = zero

    @pl.when(s == 0)
    def _():
        pltpu.sync_copy(acc_v, shared)

    plsc.subcore_barrier()

    row0 = jnp.zeros((16,), jnp.int32)
    for i in range(_CHUNK // 16):
        vals = vals_v[pl.ds(i * 16, 16)]
        idx = ids_v[pl.ds(i * 16, 16)]
        plsc.addupdate_scatter(acc_v, [row0, idx], vals)

    pltpu.sync_copy(acc_v, shared.at[idx0_v], add=True)
    plsc.subcore_barrier()

    @pl.when(s == 0)
    def _():
        pltpu.sync_copy(shared.at[0], out_hbm.at[c])


def _segsum(pa, batch, idx0):
    mesh = plsc.VectorSubcoreMesh(core_axis_name="c", subcore_axis_name="s")
    f = functools.partial(
        pl.kernel,
        mesh=mesh,
        compiler_params=pltpu.CompilerParams(needs_layout_passes=False),
        out_type=jax.ShapeDtypeStruct((_NC, _NG), jnp.float32),
        scratch_types=[
            pltpu.VMEM((_CHUNK,), jnp.float32),
            pltpu.VMEM((_CHUNK,), jnp.int32),
            pltpu.VMEM((1, _NG), jnp.float32),
            pltpu.VMEM((1,), jnp.int32),
            pltpu.VMEM_SHARED((1, _NG), jnp.float32),
        ],
    )(_segsum_body)
    return f(pa, batch, idx0)


def kernel(z, pos, batch, emb, Wp, W1, Wv, Wo1, Wo2):
    z2 = z.astype(jnp.int32).reshape(_N, 1)
    pos_pad = jnp.pad(pos, ((0, 0), (0, 5)))
    emb_pad = jnp.pad(emb, ((0, _ZPAD - _ZMAX), (0, 0)))
    Wp_pad = jnp.pad(Wp, ((0, 5), (0, 0)))
    pa = _head(z2, pos_pad, emb_pad, Wp_pad, Wv, Wo1, Wo2, W1)
    idx0 = jnp.zeros((1,), jnp.int32)
    out2 = _segsum(pa.reshape(_N), batch.astype(jnp.int32), idx0)
    return (out2[0] + out2[1]).reshape(_NG, 1)


# speedup vs baseline: 1.5726x; 1.5726x over previous
import functools

import jax
import jax.numpy as jnp
from jax import lax
from jax.experimental import pallas as pl
from jax.experimental.pallas import tpu as pltpu
from jax.experimental.pallas import tpu_sc as plsc

_N = 16384
_H = 256
_NG = 256
_ZMAX = 100
_B = 4096
_K = 2
_NH = _N // _K

_NC = 2
_NS = 16
_NW = _NC * _NS
_CHUNK = _NH // _NW

_DN0 = (((0,), (0,)), ((), ()))


def _dot0(a, b):
    return lax.dot_general(a, b, _DN0, preferred_element_type=jnp.float32)


def _head_body(z_ref, posT_ref, emb_ref, Wp_ref, Wv_ref, Wo1_ref,
               Wo2_ref, W1_ref, out_ref, embW1_ref):
    i = pl.program_id(0)

    @pl.when(i == 0)
    def _():
        embW1_ref[...] = jnp.dot(emb_ref[...], W1_ref[...],
                                 preferred_element_type=jnp.float32)

    zrow = z_ref[...].reshape(1, _B)
    ohT = (zrow == jax.lax.broadcasted_iota(jnp.int32, (_ZMAX, _B), 0)
           ).astype(jnp.bfloat16)
    posT = posT_ref[...]
    xT = _dot0(embW1_ref[...].astype(jnp.bfloat16), ohT)
    xT = xT + _dot0(Wp_ref[...], posT)
    xT = xT * jax.nn.sigmoid(xT)
    uT = _dot0(Wv_ref[...].astype(jnp.bfloat16), xT.astype(jnp.bfloat16))
    pos2 = jnp.sum(posT * posT, axis=0, keepdims=True)
    gT = xT + pos2 * (uT * uT)
    sT = _dot0(Wo1_ref[...].astype(jnp.bfloat16), gT.astype(jnp.bfloat16))
    sT = sT * jax.nn.sigmoid(sT)
    out_ref[...] = _dot0(Wo2_ref[...], sT).reshape(1, 1, _B)


def _head(z3, posT, emb, Wp, Wv, Wo1, Wo2, W1):
    nb = z3.shape[0]
    return pl.pallas_call(
        _head_body,
        grid=(nb,),
        in_specs=[
            pl.BlockSpec((1, 1, _B), lambda i: (i, 0, 0)),
            pl.BlockSpec((3, _B), lambda i: (0, i)),
            pl.BlockSpec((_ZMAX, _H), lambda i: (0, 0)),
            pl.BlockSpec((3, _H), lambda i: (0, 0)),
            pl.BlockSpec((_H, _H), lambda i: (0, 0)),
            pl.BlockSpec((_H, _H // 2), lambda i: (0, 0)),
            pl.BlockSpec((_H // 2, 1), lambda i: (0, 0)),
            pl.BlockSpec((_H, _H), lambda i: (0, 0)),
        ],
        out_specs=pl.BlockSpec((1, 1, _B), lambda i: (i, 0, 0)),
        out_shape=jax.ShapeDtypeStruct((nb, 1, _B), jnp.float32),
        scratch_shapes=[pltpu.VMEM((_ZMAX, _H), jnp.float32)],
    )(z3, posT, emb, Wp, Wv, Wo1, Wo2, W1)


def _segsum_body(pa_hbm, batch_hbm, idx0_hbm, out_hbm,
                 vals_v, ids_v, acc_v, idx0_v, shared):
    c = lax.axis_index("c")
    s = lax.axis_index("s")
    wid = s * _NC + c
    base = wid * _CHUNK
    pltpu.sync_copy(pa_hbm.at[pl.ds(base, _CHUNK)], vals_v)
    pltpu.sync_copy(batch_hbm.at[pl.ds(base, _CHUNK)], ids_v)
    pltpu.sync_copy(idx0_hbm, idx0_v)

    zero = jnp.zeros((16,), jnp.float32)
    for j in range(_NG // 16):
        acc_v[0, pl.ds(j * 16, 16)] = zero

    @pl.when(s == 0)
    def _():
        pltpu.sync_copy(acc_v, shared)

    plsc.subcore_barrier()

    row0 = jnp.zeros((16,), jnp.int32)
    for i in range(_CHUNK // 16):
        vals = vals_v[pl.ds(i * 16, 16)]
        idx = ids_v[pl.ds(i * 16, 16)]
        plsc.addupdate_scatter(acc_v, [row0, idx], vals)

    pltpu.sync_copy(acc_v, shared.at[idx0_v], add=True)
    plsc.subcore_barrier()

    @pl.when(s == 0)
    def _():
        pltpu.sync_copy(shared.at[0], out_hbm.at[c])


def _segsum(pa, batch, idx0):
    mesh = plsc.VectorSubcoreMesh(core_axis_name="c", subcore_axis_name="s")
    f = functools.partial(
        pl.kernel,
        mesh=mesh,
        compiler_params=pltpu.CompilerParams(needs_layout_passes=False),
        out_type=jax.ShapeDtypeStruct((_NC, _NG), jnp.float32),
        scratch_types=[
            pltpu.VMEM((_CHUNK,), jnp.float32),
            pltpu.VMEM((_CHUNK,), jnp.int32),
            pltpu.VMEM((1, _NG), jnp.float32),
            pltpu.VMEM((1,), jnp.int32),
            pltpu.VMEM_SHARED((1, _NG), jnp.float32),
        ],
    )(_segsum_body)
    return f(pa, batch, idx0)


def kernel(z, pos, batch, emb, Wp, W1, Wv, Wo1, Wo2):
    z3 = z.astype(jnp.int32).reshape(_N // _B, 1, _B)
    posT = pos.T
    batch32 = batch.astype(jnp.int32)
    idx0 = jnp.zeros((1,), jnp.int32)
    nbh = _NH // _B
    partials = []
    for k in range(_K):
        pa = _head(z3[k * nbh:(k + 1) * nbh], posT[:, k * _NH:(k + 1) * _NH],
                   emb, Wp, Wv, Wo1, Wo2, W1)
        partials.append(
            _segsum(pa.reshape(_NH), batch32[k * _NH:(k + 1) * _NH], idx0))
    acc = partials[0]
    for p in partials[1:]:
        acc = acc + p
    return (acc[0] + acc[1]).reshape(_NG, 1)


# speedup vs baseline: 1.6849x; 1.0714x over previous
import functools

import jax
import jax.numpy as jnp
from jax import lax
from jax.experimental import pallas as pl
from jax.experimental.pallas import tpu as pltpu
from jax.experimental.pallas import tpu_sc as plsc

_N = 16384
_H = 256
_NG = 256
_ZMAX = 100
_B = 4096
_K = 2
_NH = _N // _K

_NC = 2
_NS = 16
_NW = _NC * _NS
_CHUNK = _NH // _NW

_DN0 = (((0,), (0,)), ((), ()))


def _dot0(a, b):
    return lax.dot_general(a, b, _DN0, preferred_element_type=jnp.float32)


def _head_body(z_ref, posT_ref, emb_ref, Wp_ref, Wv_ref, Wo1_ref,
               Wo2_ref, W1_ref, out_ref, embW1_ref):
    i = pl.program_id(0)

    @pl.when(i == 0)
    def _():
        embW1_ref[...] = jnp.dot(emb_ref[...], W1_ref[...],
                                 preferred_element_type=jnp.float32)

    zrow = z_ref[...].reshape(1, _B)
    ohT = (zrow == jax.lax.broadcasted_iota(jnp.int32, (_ZMAX, _B), 0)
           ).astype(jnp.bfloat16)
    posT = posT_ref[...]
    xT = _dot0(embW1_ref[...].astype(jnp.bfloat16), ohT)
    xT = xT + _dot0(Wp_ref[...], posT)
    xT = xT * jax.nn.sigmoid(xT)
    uT = _dot0(Wv_ref[...].astype(jnp.bfloat16), xT.astype(jnp.bfloat16))
    pos2 = jnp.sum(posT * posT, axis=0, keepdims=True)
    gT = xT + pos2 * (uT * uT)
    sT = _dot0(Wo1_ref[...].astype(jnp.bfloat16), gT.astype(jnp.bfloat16))
    sT = sT * jax.nn.sigmoid(sT)
    out_ref[...] = _dot0(Wo2_ref[...], sT).reshape(1, 1, _B)


def _head(z3, posT, emb, Wp, Wv, Wo1, Wo2, W1, k):
    nbh = _NH // _B
    off = k * nbh
    return pl.pallas_call(
        _head_body,
        grid=(nbh,),
        in_specs=[
            pl.BlockSpec((1, 1, _B), lambda i: (i + off, 0, 0)),
            pl.BlockSpec((3, _B), lambda i: (0, i + off)),
            pl.BlockSpec((_ZMAX, _H), lambda i: (0, 0)),
            pl.BlockSpec((3, _H), lambda i: (0, 0)),
            pl.BlockSpec((_H, _H), lambda i: (0, 0)),
            pl.BlockSpec((_H, _H // 2), lambda i: (0, 0)),
            pl.BlockSpec((_H // 2, 1), lambda i: (0, 0)),
            pl.BlockSpec((_H, _H), lambda i: (0, 0)),
        ],
        out_specs=pl.BlockSpec((1, 1, _B), lambda i: (i, 0, 0)),
        out_shape=jax.ShapeDtypeStruct((nbh, 1, _B), jnp.float32),
        scratch_shapes=[pltpu.VMEM((_ZMAX, _H), jnp.float32)],
    )(z3, posT, emb, Wp, Wv, Wo1, Wo2, W1)


def _segsum_body(pa_hbm, batch_hbm, idx0_hbm, out_hbm,
                 vals_v, ids_v, acc_v, idx0_v, shared, *, k):
    c = lax.axis_index("c")
    s = lax.axis_index("s")
    wid = s * _NC + c
    base = wid * _CHUNK
    pltpu.sync_copy(pa_hbm.at[pl.ds(base, _CHUNK)], vals_v)
    pltpu.sync_copy(batch_hbm.at[pl.ds(k * _NH + base, _CHUNK)], ids_v)
    pltpu.sync_copy(idx0_hbm, idx0_v)

    zero = jnp.zeros((16,), jnp.float32)
    for j in range(_NG // 16):
        acc_v[0, pl.ds(j * 16, 16)] = zero

    @pl.when(s == 0)
    def _():
        pltpu.sync_copy(acc_v, shared)

    plsc.subcore_barrier()

    row0 = jnp.zeros((16,), jnp.int32)
    for i in range(_CHUNK // 16):
        vals = vals_v[pl.ds(i * 16, 16)]
        idx = ids_v[pl.ds(i * 16, 16)]
        plsc.addupdate_scatter(acc_v, [row0, idx], vals)

    pltpu.sync_copy(acc_v, shared.at[idx0_v], add=True)
    plsc.subcore_barrier()

    @pl.when(s == 0)
    def _():
        pltpu.sync_copy(shared.at[0], out_hbm.at[c])


def _segsum(pa, batch, idx0, k):
    mesh = plsc.VectorSubcoreMesh(core_axis_name="c", subcore_axis_name="s")
    f = functools.partial(
        pl.kernel,
        mesh=mesh,
        compiler_params=pltpu.CompilerParams(needs_layout_passes=False),
        out_type=jax.ShapeDtypeStruct((_NC, _NG), jnp.float32),
        scratch_types=[
            pltpu.VMEM((_CHUNK,), jnp.float32),
            pltpu.VMEM((_CHUNK,), jnp.int32),
            pltpu.VMEM((1, _NG), jnp.float32),
            pltpu.VMEM((1,), jnp.int32),
            pltpu.VMEM_SHARED((1, _NG), jnp.float32),
        ],
    )(functools.partial(_segsum_body, k=k))
    return f(pa, batch, idx0)


def kernel(z, pos, batch, emb, Wp, W1, Wv, Wo1, Wo2):
    z3 = z.astype(jnp.int32).reshape(_N // _B, 1, _B)
    posT = pos.T
    batch32 = batch.astype(jnp.int32)
    idx0 = jnp.zeros((1,), jnp.int32)
    partials = []
    for k in range(_K):
        pa = _head(z3, posT, emb, Wp, Wv, Wo1, Wo2, W1, k)
        partials.append(_segsum(pa.reshape(_NH), batch32, idx0, k))
    acc = partials[0]
    for p in partials[1:]:
        acc = acc + p
    return (acc[0] + acc[1]).reshape(_NG, 1)


# speedup vs baseline: 1.7808x; 1.0569x over previous
import functools

import jax
import jax.numpy as jnp
from jax import lax
from jax.experimental import pallas as pl
from jax.experimental.pallas import tpu as pltpu
from jax.experimental.pallas import tpu_sc as plsc

_N = 16384
_H = 256
_NG = 256
_ZMAX = 100
_B = 4096
_K = 1
_NH = _N // _K

_NC = 2
_NS = 16
_NW = _NC * _NS
_CHUNK = _NH // _NW

_DN0 = (((0,), (0,)), ((), ()))


def _dot0(a, b):
    return lax.dot_general(a, b, _DN0, preferred_element_type=jnp.float32)


def _head_body(z_ref, posT_ref, emb_ref, Wp_ref, Wv_ref, Wo1_ref,
               Wo2_ref, W1_ref, out_ref, embW1_ref):
    i = pl.program_id(0)

    @pl.when(i == 0)
    def _():
        embW1_ref[...] = jnp.dot(emb_ref[...], W1_ref[...],
                                 preferred_element_type=jnp.float32)

    zrow = z_ref[...].reshape(1, _B)
    ohT = (zrow == jax.lax.broadcasted_iota(jnp.int32, (_ZMAX, _B), 0)
           ).astype(jnp.bfloat16)
    posT = posT_ref[...]
    xT = _dot0(embW1_ref[...].astype(jnp.bfloat16), ohT)
    xT = xT + _dot0(Wp_ref[...], posT)
    xT = xT * jax.nn.sigmoid(xT)
    uT = _dot0(Wv_ref[...].astype(jnp.bfloat16), xT.astype(jnp.bfloat16))
    pos2 = jnp.sum(posT * posT, axis=0, keepdims=True)
    gT = xT + pos2 * (uT * uT)
    sT = _dot0(Wo1_ref[...].astype(jnp.bfloat16), gT.astype(jnp.bfloat16))
    sT = sT * jax.nn.sigmoid(sT)
    out_ref[...] = _dot0(Wo2_ref[...], sT).reshape(1, 1, _B)


def _head(z3, posT, emb, Wp, Wv, Wo1, Wo2, W1, k):
    nbh = _NH // _B
    off = k * nbh
    return pl.pallas_call(
        _head_body,
        grid=(nbh,),
        in_specs=[
            pl.BlockSpec((1, 1, _B), lambda i: (i + off, 0, 0)),
            pl.BlockSpec((3, _B), lambda i: (0, i + off)),
            pl.BlockSpec((_ZMAX, _H), lambda i: (0, 0)),
            pl.BlockSpec((3, _H), lambda i: (0, 0)),
            pl.BlockSpec((_H, _H), lambda i: (0, 0)),
            pl.BlockSpec((_H, _H // 2), lambda i: (0, 0)),
            pl.BlockSpec((_H // 2, 1), lambda i: (0, 0)),
            pl.BlockSpec((_H, _H), lambda i: (0, 0)),
        ],
        out_specs=pl.BlockSpec((1, 1, _B), lambda i: (i, 0, 0)),
        out_shape=jax.ShapeDtypeStruct((nbh, 1, _B), jnp.float32),
        scratch_shapes=[pltpu.VMEM((_ZMAX, _H), jnp.float32)],
    )(z3, posT, emb, Wp, Wv, Wo1, Wo2, W1)


def _segsum_body(pa_hbm, batch_hbm, idx0_hbm, out_hbm,
                 vals_v, ids_v, acc_v, idx0_v, shared, *, k):
    c = lax.axis_index("c")
    s = lax.axis_index("s")
    wid = s * _NC + c
    base = wid * _CHUNK
    pltpu.sync_copy(pa_hbm.at[pl.ds(base, _CHUNK)], vals_v)
    pltpu.sync_copy(batch_hbm.at[pl.ds(k * _NH + base, _CHUNK)], ids_v)
    pltpu.sync_copy(idx0_hbm, idx0_v)

    zero = jnp.zeros((16,), jnp.float32)
    for j in range(_NG // 16):
        acc_v[0, pl.ds(j * 16, 16)] = zero

    @pl.when(s == 0)
    def _():
        pltpu.sync_copy(acc_v, shared)

    plsc.subcore_barrier()

    row0 = jnp.zeros((16,), jnp.int32)
    for i in range(_CHUNK // 16):
        vals = vals_v[pl.ds(i * 16, 16)]
        idx = ids_v[pl.ds(i * 16, 16)]
        plsc.addupdate_scatter(acc_v, [row0, idx], vals)

    pltpu.sync_copy(acc_v, shared.at[idx0_v], add=True)
    plsc.subcore_barrier()

    @pl.when(s == 0)
    def _():
        pltpu.sync_copy(shared.at[0], out_hbm.at[c])


def _segsum(pa, batch, idx0, k):
    mesh = plsc.VectorSubcoreMesh(core_axis_name="c", subcore_axis_name="s")
    f = functools.partial(
        pl.kernel,
        mesh=mesh,
        compiler_params=pltpu.CompilerParams(needs_layout_passes=False),
        out_type=jax.ShapeDtypeStruct((_NC, _NG), jnp.float32),
        scratch_types=[
            pltpu.VMEM((_CHUNK,), jnp.float32),
            pltpu.VMEM((_CHUNK,), jnp.int32),
            pltpu.VMEM((1, _NG), jnp.float32),
            pltpu.VMEM((1,), jnp.int32),
            pltpu.VMEM_SHARED((1, _NG), jnp.float32),
        ],
    )(functools.partial(_segsum_body, k=k))
    return f(pa, batch, idx0)


def kernel(z, pos, batch, emb, Wp, W1, Wv, Wo1, Wo2):
    z3 = z.astype(jnp.int32).reshape(_N // _B, 1, _B)
    posT = pos.T
    batch32 = batch.astype(jnp.int32)
    idx0 = jnp.zeros((1,), jnp.int32)
    partials = []
    for k in range(_K):
        pa = _head(z3, posT, emb, Wp, Wv, Wo1, Wo2, W1, k)
        partials.append(_segsum(pa.reshape(_NH), batch32, idx0, k))
    acc = partials[0]
    for p in partials[1:]:
        acc = acc + p
    return (acc[0] + acc[1]).reshape(_NG, 1)


# speedup vs baseline: 1.7846x; 1.0021x over previous
import functools

import jax
import jax.numpy as jnp
from jax import lax
from jax.experimental import pallas as pl
from jax.experimental.pallas import tpu as pltpu
from jax.experimental.pallas import tpu_sc as plsc

_N = 16384
_H = 256
_NG = 256
_ZMAX = 100
_B = 8192
_K = 1
_NH = _N // _K

_NC = 2
_NS = 16
_NW = _NC * _NS
_CHUNK = _NH // _NW

_DN0 = (((0,), (0,)), ((), ()))


def _dot0(a, b):
    return lax.dot_general(a, b, _DN0, preferred_element_type=jnp.float32)


def _head_body(z_ref, posT_ref, emb_ref, Wp_ref, Wv_ref, Wo1_ref,
               Wo2_ref, W1_ref, out_ref, embW1_ref):
    i = pl.program_id(0)

    @pl.when(i == 0)
    def _():
        embW1_ref[...] = jnp.dot(emb_ref[...], W1_ref[...],
                                 preferred_element_type=jnp.float32)

    zrow = z_ref[...].reshape(1, _B)
    ohT = (zrow == jax.lax.broadcasted_iota(jnp.int32, (_ZMAX, _B), 0)
           ).astype(jnp.bfloat16)
    posT = posT_ref[...]
    xT = _dot0(embW1_ref[...].astype(jnp.bfloat16), ohT)
    xT = xT + _dot0(Wp_ref[...], posT)
    xT = xT * jax.nn.sigmoid(xT)
    uT = _dot0(Wv_ref[...].astype(jnp.bfloat16), xT.astype(jnp.bfloat16))
    pos2 = jnp.sum(posT * posT, axis=0, keepdims=True)
    gT = xT + pos2 * (uT * uT)
    sT = _dot0(Wo1_ref[...].astype(jnp.bfloat16), gT.astype(jnp.bfloat16))
    sT = sT * jax.nn.sigmoid(sT)
    out_ref[...] = _dot0(Wo2_ref[...], sT).reshape(1, 1, _B)


def _head(z3, posT, emb, Wp, Wv, Wo1, Wo2, W1, k):
    nbh = _NH // _B
    off = k * nbh
    return pl.pallas_call(
        _head_body,
        grid=(nbh,),
        in_specs=[
            pl.BlockSpec((1, 1, _B), lambda i: (i + off, 0, 0)),
            pl.BlockSpec((3, _B), lambda i: (0, i + off)),
            pl.BlockSpec((_ZMAX, _H), lambda i: (0, 0)),
            pl.BlockSpec((3, _H), lambda i: (0, 0)),
            pl.BlockSpec((_H, _H), lambda i: (0, 0)),
            pl.BlockSpec((_H, _H // 2), lambda i: (0, 0)),
            pl.BlockSpec((_H // 2, 1), lambda i: (0, 0)),
            pl.BlockSpec((_H, _H), lambda i: (0, 0)),
        ],
        out_specs=pl.BlockSpec((1, 1, _B), lambda i: (i, 0, 0)),
        out_shape=jax.ShapeDtypeStruct((nbh, 1, _B), jnp.float32),
        scratch_shapes=[pltpu.VMEM((_ZMAX, _H), jnp.float32)],
    )(z3, posT, emb, Wp, Wv, Wo1, Wo2, W1)


def _segsum_body(pa_hbm, batch_hbm, idx0_hbm, out_hbm,
                 vals_v, ids_v, acc_v, idx0_v, shared, *, k):
    c = lax.axis_index("c")
    s = lax.axis_index("s")
    wid = s * _NC + c
    base = wid * _CHUNK
    pltpu.sync_copy(pa_hbm.at[pl.ds(base, _CHUNK)], vals_v)
    pltpu.sync_copy(batch_hbm.at[pl.ds(k * _NH + base, _CHUNK)], ids_v)
    pltpu.sync_copy(idx0_hbm, idx0_v)

    zero = jnp.zeros((16,), jnp.float32)
    for j in range(_NG // 16):
        acc_v[0, pl.ds(j * 16, 16)] = zero

    @pl.when(s == 0)
    def _():
        pltpu.sync_copy(acc_v, shared)

    plsc.subcore_barrier()

    row0 = jnp.zeros((16,), jnp.int32)
    for i in range(_CHUNK // 16):
        vals = vals_v[pl.ds(i * 16, 16)]
        idx = ids_v[pl.ds(i * 16, 16)]
        plsc.addupdate_scatter(acc_v, [row0, idx], vals)

    pltpu.sync_copy(acc_v, shared.at[idx0_v], add=True)
    plsc.subcore_barrier()

    @pl.when(s == 0)
    def _():
        pltpu.sync_copy(shared.at[0], out_hbm.at[c])


def _segsum(pa, batch, idx0, k):
    mesh = plsc.VectorSubcoreMesh(core_axis_name="c", subcore_axis_name="s")
    f = functools.partial(
        pl.kernel,
        mesh=mesh,
        compiler_params=pltpu.CompilerParams(needs_layout_passes=False),
        out_type=jax.ShapeDtypeStruct((_NC, _NG), jnp.float32),
        scratch_types=[
            pltpu.VMEM((_CHUNK,), jnp.float32),
            pltpu.VMEM((_CHUNK,), jnp.int32),
            pltpu.VMEM((1, _NG), jnp.float32),
            pltpu.VMEM((1,), jnp.int32),
            pltpu.VMEM_SHARED((1, _NG), jnp.float32),
        ],
    )(functools.partial(_segsum_body, k=k))
    return f(pa, batch, idx0)


def kernel(z, pos, batch, emb, Wp, W1, Wv, Wo1, Wo2):
    z3 = z.astype(jnp.int32).reshape(_N // _B, 1, _B)
    posT = pos.T
    batch32 = batch.astype(jnp.int32)
    idx0 = jnp.zeros((1,), jnp.int32)
    partials = []
    for k in range(_K):
        pa = _head(z3, posT, emb, Wp, Wv, Wo1, Wo2, W1, k)
        partials.append(_segsum(pa.reshape(_NH), batch32, idx0, k))
    acc = partials[0]
    for p in partials[1:]:
        acc = acc + p
    return (acc[0] + acc[1]).reshape(_NG, 1)
